# Initial kernel scaffold; baseline (speedup 1.0000x reference)
import jax
import jax.numpy as jnp
from jax.experimental import pallas as pl


def kernel(x, edge_idx, edge_attr, W1a, b1a, g1, be1, W1b, b1b, W2a, b2a, g2, be2, W2b, b2b):
    raise NotImplementedError("write your pallas kernel here")


# speedup vs baseline: 2.6676x; 2.6676x over previous
import functools

import jax
import jax.numpy as jnp
from jax import lax
from jax.experimental import pallas as pl
from jax.experimental.pallas import tpu as pltpu
from jax.experimental.pallas import tpu_sc as plsc

N, E, F, H, T = 10000, 320000, 128, 64, 64
NC, NS = 2, 16
NW = NC * NS
EPW = E // NW
CH = 80
NCH = EPW // CH
STRIPE = N // NS
CW = 8

_mesh = plsc.VectorSubcoreMesh(core_axis_name="c", subcore_axis_name="s",
                               num_cores=NC, num_subcores=NS)
_sc_params = pltpu.CompilerParams(use_tc_tiling_on_sc=False)


@functools.partial(
    pl.kernel,
    out_type=jax.ShapeDtypeStruct((NW, NCH, CH, H), jnp.float32),
    mesh=_mesh,
    compiler_params=_sc_params,
    scratch_types=[
        pltpu.VMEM((NCH, CH), jnp.int32),
        pltpu.VMEM((CH, H), jnp.float32),
        pltpu.SemaphoreType.DMA,
    ],
)
def _sc_gather(xa_hbm, src_hbm, gx_hbm, idx_v, rows_v, sem):
    wid = lax.axis_index("c") * NS + lax.axis_index("s")
    pltpu.sync_copy(src_hbm.at[wid], idx_v)

    @pl.loop(0, NCH)
    def _chunk(j):
        pltpu.async_copy(xa_hbm.at[idx_v.at[j]], rows_v, sem).wait()
        pltpu.sync_copy(rows_v, gx_hbm.at[wid, j])


@functools.partial(
    pl.kernel,
    out_type=(jax.ShapeDtypeStruct((NC, N, H), jnp.float32),
              jax.ShapeDtypeStruct((NC, N, CW), jnp.float32)),
    mesh=_mesh,
    compiler_params=_sc_params,
    scratch_types=[
        pltpu.VMEM((NCH, CH), jnp.int32),
        pltpu.VMEM((CH, H), jnp.float32),
        pltpu.VMEM((CH, CW), jnp.float32),
        pltpu.VMEM_SHARED((N, H), jnp.float32),
        pltpu.VMEM_SHARED((N, CW), jnp.float32),
    ],
)
def _sc_scatter(h_hbm, dst_hbm, zs_hbm, zc_hbm, ones_hbm, s_out, c_out,
                idx_v, hv, ones_v, s_sh, c_sh):
    c = lax.axis_index("c")
    s = lax.axis_index("s")
    wid = c * NS + s
    pltpu.sync_copy(zs_hbm, s_sh.at[pl.ds(s * STRIPE, STRIPE)])
    pltpu.sync_copy(zc_hbm, c_sh.at[pl.ds(s * STRIPE, STRIPE)])
    pltpu.sync_copy(ones_hbm, ones_v)
    pltpu.sync_copy(dst_hbm.at[wid], idx_v)
    plsc.subcore_barrier()

    @pl.loop(0, NCH)
    def _chunk(j):
        pltpu.sync_copy(h_hbm.at[wid, j], hv)
        pltpu.sync_copy(hv, s_sh.at[idx_v.at[j]], add=True)
        pltpu.sync_copy(ones_v, c_sh.at[idx_v.at[j]], add=True)

    plsc.subcore_barrier()
    pltpu.sync_copy(s_sh.at[pl.ds(s * STRIPE, STRIPE)],
                    s_out.at[c, pl.ds(s * STRIPE, STRIPE)])
    pltpu.sync_copy(c_sh.at[pl.ds(s * STRIPE, STRIPE)],
                    c_out.at[c, pl.ds(s * STRIPE, STRIPE)])


def _xa_body(x_ref, w_ref, o_ref):
    o_ref[...] = jnp.dot(x_ref[...], w_ref[...],
                         preferred_element_type=jnp.float32)


def _edge_body(gx_ref, ea_ref, w_ref, b_ref, g_ref, be_ref, o_ref):
    t = gx_ref[...] + jnp.dot(ea_ref[...], w_ref[...],
                              preferred_element_type=jnp.float32) + b_ref[...]
    t = jnp.where(t >= 0, t, 0.01 * t)
    m = jnp.mean(t, axis=-1, keepdims=True)
    v = jnp.mean((t - m) ** 2, axis=-1, keepdims=True)
    o_ref[...] = (t - m) * lax.rsqrt(v + 1e-5) * g_ref[...] + be_ref[...]


def _node_body(x_ref, sp_ref, cp_ref, w1b_ref, b1b_ref, w2x_ref, w2a_ref,
               b2a_ref, g2_ref, be2_ref, w2b_ref, b2b_ref, o_ref):
    ssum = sp_ref[0] + sp_ref[1]
    cnt = cp_ref[0, :, 0:1] + cp_ref[1, :, 0:1]
    hbar = ssum / jnp.maximum(cnt, 1.0)
    agg = jnp.dot(hbar, w1b_ref[...], preferred_element_type=jnp.float32) \
        + b1b_ref[...]
    agg = jnp.where(cnt > 0, agg, 0.0)
    t = jnp.dot(x_ref[...], w2x_ref[...], preferred_element_type=jnp.float32) \
        + jnp.dot(agg, w2a_ref[...], preferred_element_type=jnp.float32) \
        + b2a_ref[...]
    t = jnp.where(t >= 0, t, 0.01 * t)
    m = jnp.mean(t, axis=-1, keepdims=True)
    v = jnp.mean((t - m) ** 2, axis=-1, keepdims=True)
    t = (t - m) * lax.rsqrt(v + 1e-5) * g2_ref[...] + be2_ref[...]
    o_ref[...] = jnp.dot(t, w2b_ref[...], preferred_element_type=jnp.float32) \
        + b2b_ref[...]


_BN = 2000
_BE = 4000


def _const_spec(shape):
    nd = len(shape)
    return pl.BlockSpec(shape, lambda i: (0,) * nd)


def kernel(x, edge_idx, edge_attr, W1a, b1a, g1, be1, W1b, b1b,
           W2a, b2a, g2, be2, W2b, b2b):
    src = edge_idx[0].reshape(NW, NCH, CH)
    dst = edge_idx[1].reshape(NW, NCH, CH)
    w1x = W1a[:, :F].T
    w1e = W1a[:, F:].T
    zs = jnp.zeros((STRIPE, H), jnp.float32)
    zc = jnp.zeros((STRIPE, CW), jnp.float32)
    ones = jnp.ones((CH, CW), jnp.float32)

    xa = pl.pallas_call(
        _xa_body,
        grid=(N // _BN,),
        in_specs=[pl.BlockSpec((_BN, F), lambda i: (i, 0)),
                  _const_spec((F, H))],
        out_specs=pl.BlockSpec((_BN, H), lambda i: (i, 0)),
        out_shape=jax.ShapeDtypeStruct((N, H), jnp.float32),
    )(x, w1x)

    gx = _sc_gather(xa, src).reshape(E, H)

    h = pl.pallas_call(
        _edge_body,
        grid=(E // _BE,),
        in_specs=[pl.BlockSpec((_BE, H), lambda i: (i, 0)),
                  pl.BlockSpec((_BE, H), lambda i: (i, 0)),
                  _const_spec((H, H)), _const_spec((1, H)),
                  _const_spec((1, H)), _const_spec((1, H))],
        out_specs=pl.BlockSpec((_BE, H), lambda i: (i, 0)),
        out_shape=jax.ShapeDtypeStruct((E, H), jnp.float32),
    )(gx, edge_attr, w1e, b1a[None], g1[None], be1[None])

    s_parts, c_parts = _sc_scatter(h.reshape(NW, NCH, CH, H), dst,
                                   zs, zc, ones)

    o = pl.pallas_call(
        _node_body,
        grid=(N // _BN,),
        in_specs=[pl.BlockSpec((_BN, F), lambda i: (i, 0)),
                  pl.BlockSpec((NC, _BN, H), lambda i: (0, i, 0)),
                  pl.BlockSpec((NC, _BN, CW), lambda i: (0, i, 0)),
                  _const_spec((H, H)), _const_spec((1, H)),
                  _const_spec((F, H)), _const_spec((H, H)),
                  _const_spec((1, H)), _const_spec((1, H)),
                  _const_spec((1, H)), _const_spec((H, T)),
                  _const_spec((1, T))],
        out_specs=pl.BlockSpec((_BN, T), lambda i: (i, 0)),
        out_shape=jax.ShapeDtypeStruct((N, T), jnp.float32),
    )(x, s_parts, c_parts, W1b.T, b1b[None], W2a[:, :F].T, W2a[:, F:].T,
      b2a[None], g2[None], be2[None], W2b.T, b2b[None])
    return o


# speedup vs baseline: 3.0510x; 1.1437x over previous
import functools

import jax
import jax.numpy as jnp
from jax import lax
from jax.experimental import pallas as pl
from jax.experimental.pallas import tpu as pltpu
from jax.experimental.pallas import tpu_sc as plsc

N, E, F, H, T = 10000, 320000, 128, 64, 64
NC, NS = 2, 16
NW = NC * NS
EPW = E // NW
CH = 100
NCH = EPW // CH
STRIPE = N // NS
CW = 8

_mesh = plsc.VectorSubcoreMesh(core_axis_name="c", subcore_axis_name="s",
                               num_cores=NC, num_subcores=NS)
_sc_params = pltpu.CompilerParams(use_tc_tiling_on_sc=False)


@functools.partial(
    pl.kernel,
    out_type=jax.ShapeDtypeStruct((NW, NCH, CH, H), jnp.float32),
    mesh=_mesh,
    compiler_params=_sc_params,
    scratch_types=[
        pltpu.VMEM((NCH, CH), jnp.int32),
        pltpu.VMEM((CH, H), jnp.float32),
        pltpu.VMEM((CH, H), jnp.float32),
        pltpu.SemaphoreType.DMA,
        pltpu.SemaphoreType.DMA,
    ],
)
def _sc_gather(xa_hbm, src_hbm, gx_hbm, idx_v, rows0, rows1, sem0, sem1):
    wid = lax.axis_index("c") * NS + lax.axis_index("s")
    pltpu.sync_copy(src_hbm.at[wid], idx_v)
    pltpu.async_copy(xa_hbm.at[idx_v.at[0]], rows0, sem0)

    @pl.loop(0, NCH, step=2)
    def _chunk(j):
        pltpu.async_copy(xa_hbm.at[idx_v.at[j + 1]], rows1, sem1)
        pltpu.make_async_copy(xa_hbm.at[idx_v.at[j]], rows0, sem0).wait()
        pltpu.sync_copy(rows0, gx_hbm.at[wid, j])

        @pl.when(j + 2 < NCH)
        def _():
            pltpu.async_copy(xa_hbm.at[idx_v.at[j + 2]], rows0, sem0)

        pltpu.make_async_copy(xa_hbm.at[idx_v.at[j]], rows1, sem1).wait()
        pltpu.sync_copy(rows1, gx_hbm.at[wid, j + 1])


@functools.partial(
    pl.kernel,
    out_type=(jax.ShapeDtypeStruct((NC, N, H), jnp.float32),
              jax.ShapeDtypeStruct((NC, N, CW), jnp.float32)),
    mesh=_mesh,
    compiler_params=_sc_params,
    scratch_types=[
        pltpu.VMEM((NCH, CH), jnp.int32),
        pltpu.VMEM((CH, H), jnp.float32),
        pltpu.VMEM((CH, H), jnp.float32),
        pltpu.VMEM((CH, CW), jnp.float32),
        pltpu.VMEM_SHARED((N, H), jnp.float32),
        pltpu.VMEM_SHARED((N, CW), jnp.float32),
        pltpu.SemaphoreType.DMA,
        pltpu.SemaphoreType.DMA,
    ],
)
def _sc_scatter(h_hbm, dst_hbm, zs_hbm, zc_hbm, ones_hbm, s_out, c_out,
                idx_v, hv0, hv1, ones_v, s_sh, c_sh, sem0, sem1):
    c = lax.axis_index("c")
    s = lax.axis_index("s")
    wid = c * NS + s
    pltpu.sync_copy(zs_hbm, s_sh.at[pl.ds(s * STRIPE, STRIPE)])
    pltpu.sync_copy(zc_hbm, c_sh.at[pl.ds(s * STRIPE, STRIPE)])
    pltpu.sync_copy(ones_hbm, ones_v)
    pltpu.sync_copy(dst_hbm.at[wid], idx_v)
    plsc.subcore_barrier()
    pltpu.async_copy(h_hbm.at[wid, 0], hv0, sem0)

    @pl.loop(0, NCH, step=2)
    def _chunk(j):
        pltpu.async_copy(h_hbm.at[wid, j + 1], hv1, sem1)
        pltpu.make_async_copy(h_hbm.at[wid, j], hv0, sem0).wait()
        pltpu.sync_copy(hv0, s_sh.at[idx_v.at[j]], add=True)
        pltpu.sync_copy(ones_v, c_sh.at[idx_v.at[j]], add=True)

        @pl.when(j + 2 < NCH)
        def _():
            pltpu.async_copy(h_hbm.at[wid, j + 2], hv0, sem0)

        pltpu.make_async_copy(h_hbm.at[wid, j], hv1, sem1).wait()
        pltpu.sync_copy(hv1, s_sh.at[idx_v.at[j + 1]], add=True)
        pltpu.sync_copy(ones_v, c_sh.at[idx_v.at[j + 1]], add=True)

    plsc.subcore_barrier()
    pltpu.sync_copy(s_sh.at[pl.ds(s * STRIPE, STRIPE)],
                    s_out.at[c, pl.ds(s * STRIPE, STRIPE)])
    pltpu.sync_copy(c_sh.at[pl.ds(s * STRIPE, STRIPE)],
                    c_out.at[c, pl.ds(s * STRIPE, STRIPE)])


def _xa_body(x_ref, w_ref, o_ref):
    o_ref[...] = jnp.dot(x_ref[...], w_ref[...],
                         preferred_element_type=jnp.float32)


def _edge_body(gx_ref, ea_ref, w_ref, b_ref, g_ref, be_ref, o_ref):
    t = gx_ref[...] + jnp.dot(ea_ref[...], w_ref[...],
                              preferred_element_type=jnp.float32) + b_ref[...]
    t = jnp.maximum(t, 0.01 * t)
    m = jnp.sum(t, axis=-1, keepdims=True) * (1.0 / H)
    v = jnp.sum(t * t, axis=-1, keepdims=True) * (1.0 / H) - m * m
    o_ref[...] = (t - m) * (lax.rsqrt(v + 1e-5) * g_ref[...]) + be_ref[...]


def _node_body(x_ref, sp_ref, cp_ref, w1b_ref, b1b_ref, w2x_ref, w2a_ref,
               b2a_ref, g2_ref, be2_ref, w2b_ref, b2b_ref, o_ref):
    ssum = sp_ref[0] + sp_ref[1]
    cnt = cp_ref[0, :, 0:1] + cp_ref[1, :, 0:1]
    hbar = ssum / jnp.maximum(cnt, 1.0)
    agg = jnp.dot(hbar, w1b_ref[...], preferred_element_type=jnp.float32) \
        + b1b_ref[...]
    agg = jnp.where(cnt > 0, agg, 0.0)
    t = jnp.dot(x_ref[...], w2x_ref[...], preferred_element_type=jnp.float32) \
        + jnp.dot(agg, w2a_ref[...], preferred_element_type=jnp.float32) \
        + b2a_ref[...]
    t = jnp.where(t >= 0, t, 0.01 * t)
    m = jnp.mean(t, axis=-1, keepdims=True)
    v = jnp.mean((t - m) ** 2, axis=-1, keepdims=True)
    t = (t - m) * lax.rsqrt(v + 1e-5) * g2_ref[...] + be2_ref[...]
    o_ref[...] = jnp.dot(t, w2b_ref[...], preferred_element_type=jnp.float32) \
        + b2b_ref[...]


_BN = 2000
_BE = 4000


def _const_spec(shape):
    nd = len(shape)
    return pl.BlockSpec(shape, lambda i: (0,) * nd)


def kernel(x, edge_idx, edge_attr, W1a, b1a, g1, be1, W1b, b1b,
           W2a, b2a, g2, be2, W2b, b2b):
    src = edge_idx[0].reshape(NW, NCH, CH)
    dst = edge_idx[1].reshape(NW, NCH, CH)
    w1x = W1a[:, :F].T
    w1e = W1a[:, F:].T
    zs = jnp.zeros((STRIPE, H), jnp.float32)
    zc = jnp.zeros((STRIPE, CW), jnp.float32)
    ones = jnp.ones((CH, CW), jnp.float32)

    xa = pl.pallas_call(
        _xa_body,
        grid=(N // _BN,),
        in_specs=[pl.BlockSpec((_BN, F), lambda i: (i, 0)),
                  _const_spec((F, H))],
        out_specs=pl.BlockSpec((_BN, H), lambda i: (i, 0)),
        out_shape=jax.ShapeDtypeStruct((N, H), jnp.float32),
    )(x, w1x)

    gx = _sc_gather(xa, src).reshape(E, H)

    h = pl.pallas_call(
        _edge_body,
        grid=(E // _BE,),
        in_specs=[pl.BlockSpec((_BE, H), lambda i: (i, 0)),
                  pl.BlockSpec((_BE, H), lambda i: (i, 0)),
                  _const_spec((H, H)), _const_spec((1, H)),
                  _const_spec((1, H)), _const_spec((1, H))],
        out_specs=pl.BlockSpec((_BE, H), lambda i: (i, 0)),
        out_shape=jax.ShapeDtypeStruct((E, H), jnp.float32),
    )(gx, edge_attr, w1e, b1a[None], g1[None], be1[None])

    s_parts, c_parts = _sc_scatter(h.reshape(NW, NCH, CH, H), dst,
                                   zs, zc, ones)

    o = pl.pallas_call(
        _node_body,
        grid=(N // _BN,),
        in_specs=[pl.BlockSpec((_BN, F), lambda i: (i, 0)),
                  pl.BlockSpec((NC, _BN, H), lambda i: (0, i, 0)),
                  pl.BlockSpec((NC, _BN, CW), lambda i: (0, i, 0)),
                  _const_spec((H, H)), _const_spec((1, H)),
                  _const_spec((F, H)), _const_spec((H, H)),
                  _const_spec((1, H)), _const_spec((1, H)),
                  _const_spec((1, H)), _const_spec((H, T)),
                  _const_spec((1, T))],
        out_specs=pl.BlockSpec((_BN, T), lambda i: (i, 0)),
        out_shape=jax.ShapeDtypeStruct((N, T), jnp.float32),
    )(x, s_parts, c_parts, W1b.T, b1b[None], W2a[:, :F].T, W2a[:, F:].T,
      b2a[None], g2[None], be2[None], W2b.T, b2b[None])
    return o


# speedup vs baseline: 3.6597x; 1.1995x over previous
import functools

import jax
import jax.numpy as jnp
from jax import lax
from jax.experimental import pallas as pl
from jax.experimental.pallas import tpu as pltpu
from jax.experimental.pallas import tpu_sc as plsc

N, E, F, H, T = 10000, 320000, 128, 64, 64
NC, NS = 2, 16
NW = NC * NS
EPW = E // NW
CH = 100
NCH = EPW // CH
E2 = E // 2
EBW = EPW // 2
EBC = CH // 2
STRIPE = N // NS
CW = 8

_mesh = plsc.VectorSubcoreMesh(core_axis_name="c", subcore_axis_name="s",
                               num_cores=NC, num_subcores=NS)
_sc_params = pltpu.CompilerParams(use_tc_tiling_on_sc=False)


def _rsqrt16(x):
    xh = x * 0.5
    i = lax.bitcast_convert_type(x, jnp.int32)
    i = jnp.int32(0x5F3759DF) - lax.shift_right_logical(i, 1)
    y = lax.bitcast_convert_type(i, jnp.float32)
    y = y * (1.5 - xh * y * y)
    y = y * (1.5 - xh * y * y)
    y = y * (1.5 - xh * y * y)
    return y


def _perm16(x, idx):
    return lax.gather(
        x, idx[:, None],
        lax.GatherDimensionNumbers(offset_dims=(), collapsed_slice_dims=(0,),
                                   start_index_map=(0,)),
        slice_sizes=(1,),
        mode=lax.GatherScatterMode.PROMISE_IN_BOUNDS)


def _splat_sum16(x):
    for s in (8, 4, 2, 1):
        idx = jnp.bitwise_xor(lax.iota(jnp.int32, 16), s)
        x = x + _perm16(x, idx)
    return x


@functools.partial(
    pl.kernel,
    out_type=(jax.ShapeDtypeStruct((NC, N, H), jnp.float32),
              jax.ShapeDtypeStruct((NC, N, CW), jnp.float32)),
    mesh=_mesh,
    compiler_params=_sc_params,
    scratch_types=[
        pltpu.VMEM((NCH, CH), jnp.int32),
        pltpu.VMEM((NCH, CH), jnp.int32),
        pltpu.VMEM((EBC, 128), jnp.float32),
        pltpu.VMEM((EBC, 128), jnp.float32),
        pltpu.VMEM((CH, H), jnp.float32),
        pltpu.VMEM((CH, H), jnp.float32),
        pltpu.VMEM((CH, H), jnp.float32),
        pltpu.VMEM((CH, CW), jnp.float32),
        pltpu.VMEM((2, H), jnp.float32),
        pltpu.VMEM_SHARED((N, H), jnp.float32),
        pltpu.VMEM_SHARED((N, CW), jnp.float32),
        pltpu.SemaphoreType.DMA,
        pltpu.SemaphoreType.DMA,
        pltpu.SemaphoreType.DMA,
        pltpu.SemaphoreType.DMA,
    ],
)
def _sc_fused(xa_hbm, src_hbm, dst_hbm, eb_hbm, gbe_hbm, zs_hbm, zc_hbm,
              ones_hbm, s_out, c_out,
              idx_s, idx_d, ebv0, ebv1, gxv0, gxv1, hv, ones_v, gbe_v,
              s_sh, c_sh, se0, se1, sg0, sg1):
    c = lax.axis_index("c")
    s = lax.axis_index("s")
    wid = c * NS + s
    ebbase = wid * EBW

    pltpu.sync_copy(src_hbm.at[wid], idx_s)
    pltpu.sync_copy(dst_hbm.at[wid], idx_d)
    pltpu.sync_copy(ones_hbm, ones_v)
    pltpu.sync_copy(gbe_hbm, gbe_v)
    pltpu.sync_copy(zs_hbm, s_sh.at[pl.ds(s * STRIPE, STRIPE)])
    pltpu.sync_copy(zc_hbm, c_sh.at[pl.ds(s * STRIPE, STRIPE)])
    plsc.subcore_barrier()

    gk = [gbe_v[0, pl.ds(16 * k, 16)] for k in range(4)]
    bek = [gbe_v[1, pl.ds(16 * k, 16)] for k in range(4)]

    def fire(j, ebv, gxv, sem_e, sem_g):
        pltpu.async_copy(eb_hbm.at[pl.ds(ebbase + j * EBC, EBC)], ebv, sem_e)
        pltpu.async_copy(xa_hbm.at[idx_s.at[j]], gxv, sem_g)

    def wait(j, ebv, gxv, sem_e, sem_g):
        pltpu.make_async_copy(eb_hbm.at[pl.ds(ebbase, EBC)], ebv, sem_e).wait()
        pltpu.make_async_copy(xa_hbm.at[idx_s.at[j]], gxv, sem_g).wait()

    def compute_and_scatter(j, ebv, gxv):
        @pl.loop(0, EBC)
        def _row(r):
            for half in range(2):
                e = 2 * r + half
                ofs = half * H
                t = [gxv[e, pl.ds(16 * k, 16)]
                     + ebv[r, pl.ds(ofs + 16 * k, 16)] for k in range(4)]
                t = [jnp.maximum(tk, 0.01 * tk) for tk in t]
                mv = _splat_sum16(t[0] + t[1] + t[2] + t[3]) * (1.0 / H)
                qv = _splat_sum16(t[0] * t[0] + t[1] * t[1]
                                  + t[2] * t[2] + t[3] * t[3]) * (1.0 / H)
                rv = _rsqrt16(qv - mv * mv + 1e-5)
                for k in range(4):
                    hv[e, pl.ds(16 * k, 16)] = \
                        (t[k] - mv) * (rv * gk[k]) + bek[k]

        pltpu.sync_copy(hv, s_sh.at[idx_d.at[j]], add=True)
        pltpu.sync_copy(ones_v, c_sh.at[idx_d.at[j]], add=True)

    fire(0, ebv0, gxv0, se0, sg0)

    @pl.loop(0, NCH, step=2)
    def _chunk(j):
        fire(j + 1, ebv1, gxv1, se1, sg1)
        wait(j, ebv0, gxv0, se0, sg0)
        compute_and_scatter(j, ebv0, gxv0)

        @pl.when(j + 2 < NCH)
        def _():
            fire(j + 2, ebv0, gxv0, se0, sg0)

        wait(j + 1, ebv1, gxv1, se1, sg1)
        compute_and_scatter(j + 1, ebv1, gxv1)

    plsc.subcore_barrier()
    pltpu.sync_copy(s_sh.at[pl.ds(s * STRIPE, STRIPE)],
                    s_out.at[c, pl.ds(s * STRIPE, STRIPE)])
    pltpu.sync_copy(c_sh.at[pl.ds(s * STRIPE, STRIPE)],
                    c_out.at[c, pl.ds(s * STRIPE, STRIPE)])


def _xa_body(x_ref, w_ref, o_ref):
    o_ref[...] = jnp.dot(x_ref[...], w_ref[...],
                         preferred_element_type=jnp.float32)


def _eb_body(ea_lo_ref, ea_hi_ref, w_ref, b_ref, o_ref):
    lo = jnp.dot(ea_lo_ref[...], w_ref[...],
                 preferred_element_type=jnp.float32) + b_ref[...]
    hi = jnp.dot(ea_hi_ref[...], w_ref[...],
                 preferred_element_type=jnp.float32) + b_ref[...]
    o_ref[...] = jnp.concatenate([lo, hi], axis=1)


def _node_body(x_ref, sp_ref, cp_ref, w1b_ref, b1b_ref, w2x_ref, w2a_ref,
               b2a_ref, g2_ref, be2_ref, w2b_ref, b2b_ref, o_ref):
    ssum = sp_ref[0] + sp_ref[1]
    cnt = cp_ref[0, :, 0:1] + cp_ref[1, :, 0:1]
    hbar = ssum / jnp.maximum(cnt, 1.0)
    agg = jnp.dot(hbar, w1b_ref[...], preferred_element_type=jnp.float32) \
        + b1b_ref[...]
    agg = jnp.where(cnt > 0, agg, 0.0)
    t = jnp.dot(x_ref[...], w2x_ref[...], preferred_element_type=jnp.float32) \
        + jnp.dot(agg, w2a_ref[...], preferred_element_type=jnp.float32) \
        + b2a_ref[...]
    t = jnp.maximum(t, 0.01 * t)
    m = jnp.sum(t, axis=-1, keepdims=True) * (1.0 / H)
    v = jnp.sum(t * t, axis=-1, keepdims=True) * (1.0 / H) - m * m
    t = (t - m) * (lax.rsqrt(v + 1e-5) * g2_ref[...]) + be2_ref[...]
    o_ref[...] = jnp.dot(t, w2b_ref[...], preferred_element_type=jnp.float32) \
        + b2b_ref[...]


_BN = 2000
_BE = 4000


def _const_spec(shape):
    nd = len(shape)
    return pl.BlockSpec(shape, lambda i: (0,) * nd)


def kernel(x, edge_idx, edge_attr, W1a, b1a, g1, be1, W1b, b1b,
           W2a, b2a, g2, be2, W2b, b2b):
    src = jnp.stack([edge_idx[0, :E2], edge_idx[0, E2:]], axis=1) \
        .reshape(NW, NCH, CH)
    dst = jnp.stack([edge_idx[1, :E2], edge_idx[1, E2:]], axis=1) \
        .reshape(NW, NCH, CH)
    w1x = W1a[:, :F].T
    w1e = W1a[:, F:].T
    gbe = jnp.stack([g1, be1])
    zs = jnp.zeros((STRIPE, H), jnp.float32)
    zc = jnp.zeros((STRIPE, CW), jnp.float32)
    ones = jnp.ones((CH, CW), jnp.float32)

    xa = pl.pallas_call(
        _xa_body,
        grid=(N // _BN,),
        in_specs=[pl.BlockSpec((_BN, F), lambda i: (i, 0)),
                  _const_spec((F, H))],
        out_specs=pl.BlockSpec((_BN, H), lambda i: (i, 0)),
        out_shape=jax.ShapeDtypeStruct((N, H), jnp.float32),
    )(x, w1x)

    nblk = E2 // _BE
    eb = pl.pallas_call(
        _eb_body,
        grid=(nblk,),
        in_specs=[pl.BlockSpec((_BE, H), lambda i: (i, 0)),
                  pl.BlockSpec((_BE, H), lambda i: (i + nblk, 0)),
                  _const_spec((H, H)), _const_spec((1, H))],
        out_specs=pl.BlockSpec((_BE, 128), lambda i: (i, 0)),
        out_shape=jax.ShapeDtypeStruct((E2, 128), jnp.float32),
    )(edge_attr, edge_attr, w1e, b1a[None])

    s_parts, c_parts = _sc_fused(xa, src, dst, eb, gbe, zs, zc, ones)

    o = pl.pallas_call(
        _node_body,
        grid=(N // _BN,),
        in_specs=[pl.BlockSpec((_BN, F), lambda i: (i, 0)),
                  pl.BlockSpec((NC, _BN, H), lambda i: (0, i, 0)),
                  pl.BlockSpec((NC, _BN, CW), lambda i: (0, i, 0)),
                  _const_spec((H, H)), _const_spec((1, H)),
                  _const_spec((F, H)), _const_spec((H, H)),
                  _const_spec((1, H)), _const_spec((1, H)),
                  _const_spec((1, H)), _const_spec((H, T)),
                  _const_spec((1, T))],
        out_specs=pl.BlockSpec((_BN, T), lambda i: (i, 0)),
        out_shape=jax.ShapeDtypeStruct((N, T), jnp.float32),
    )(x, s_parts, c_parts, W1b.T, b1b[None], W2a[:, :F].T, W2a[:, F:].T,
      b2a[None], g2[None], be2[None], W2b.T, b2b[None])
    return o
